# Initial kernel scaffold; baseline (speedup 1.0000x reference)
import jax
import jax.numpy as jnp
from jax.experimental import pallas as pl


def kernel(qs_wt, aa_wt, aa_mut, agent_mask, mask, mutation_mask, emb_wt, emb_mut, w1a, b1a, w1b, b1b, wfa, bfa, wfb, bfb, hb1_w, hb1_b, va_w, va_b, vb_w, vb_b):
    raise NotImplementedError("write your pallas kernel here")


# speedup vs baseline: 2.5724x; 2.5724x over previous
import functools

import jax
import jax.numpy as jnp
from jax import lax
from jax.experimental import pallas as pl
from jax.experimental.pallas import tpu as pltpu
from jax.experimental.pallas import tpu_sc as plsc

N_AG = 20
BINS = 32
T = 64
EMBED_DIM = 32


def _sc_body(qs_hbm, aaw_hbm, aam_hbm, am_hbm, m_hbm, mm_hbm, ew_hbm, em_hbm,
             zeros_hbm,
             qsel_hbm, gs_hbm, qparts_hbm, cparts_hbm,
             qs_buf, ew_buf, em_buf,
             aw_buf, amut_buf, amk_buf, mk_buf, mmk_buf,
             eidxw_buf, eidxm_buf, qout_buf, qacc, cacc, eacc, spacc):
    B, L = aaw_hbm.shape
    s = lax.axis_index("s")
    c = lax.axis_index("c")
    lb = s // 2
    h = s % 2
    b = c * (B // 2) + lb
    tok_per_tile = L // 2
    tok_base = h * tok_per_tile
    nchunks = tok_per_tile // T

    for j in range(BINS):
        qacc[pl.ds(j * 16, 16)] = jnp.zeros((16,), jnp.float32)
        cacc[pl.ds(j * 16, 16)] = jnp.zeros((16,), jnp.float32)

    pltpu.sync_copy(zeros_hbm, spacc.at[pl.ds(s * BINS, BINS)])

    lanes = lax.iota(jnp.int32, 16)
    trash = s * BINS + N_AG + (lanes & 7)

    def chunk(i, _):
        t0 = tok_base + i * T
        pltpu.sync_copy(qs_hbm.at[b, pl.ds(t0 * N_AG, T * N_AG)], qs_buf)
        pltpu.sync_copy(aaw_hbm.at[b, pl.ds(t0, T)], aw_buf)
        pltpu.sync_copy(aam_hbm.at[b, pl.ds(t0, T)], amut_buf)
        pltpu.sync_copy(am_hbm.at[b, pl.ds(t0, T)], amk_buf)
        pltpu.sync_copy(m_hbm.at[b, pl.ds(t0, T)], mk_buf)
        pltpu.sync_copy(mm_hbm.at[b, pl.ds(t0, T)], mmk_buf)
        pltpu.sync_copy(ew_hbm.at[b, pl.ds(t0, T)], ew_buf)
        pltpu.sync_copy(em_hbm.at[b, pl.ds(t0, T)], em_buf)
        for k in range(T // 16):
            sl = pl.ds(k * 16, 16)
            aw = aw_buf[sl]
            amut = amut_buf[sl]
            valid = (amk_buf[sl] & mk_buf[sl]) != 0
            sel = jnp.where(mmk_buf[sl] != 0, amut, aw)
            q = plsc.load_gather(qs_buf, [(lanes + (k * 16)) * N_AG + sel])
            qv = jnp.where(valid, q, 0.0)
            qout_buf[pl.ds(i * T + k * 16, 16)] = qv
            plsc.addupdate_scatter(qacc, [aw * 16 + lanes], qv)
            ones = jnp.where(valid, 1.0, 0.0)
            plsc.addupdate_scatter(cacc, [aw * 16 + lanes], ones)
            plsc.addupdate_scatter(cacc, [amut * 16 + lanes], ones)
            eidxw_buf[sl] = jnp.where(valid, s * BINS + aw, trash)
            eidxm_buf[sl] = jnp.where(valid, s * BINS + amut, trash)
        pltpu.sync_copy(ew_buf, spacc.at[eidxw_buf], add=True)
        pltpu.sync_copy(em_buf, spacc.at[eidxm_buf], add=True)
        return 0

    lax.fori_loop(0, nchunks, chunk, 0)

    pltpu.sync_copy(qout_buf, qsel_hbm.at[b, pl.ds(tok_base, tok_per_tile)])
    pltpu.sync_copy(qacc, qparts_hbm.at[b, h])
    pltpu.sync_copy(cacc, cparts_hbm.at[b, h])
    pltpu.sync_copy(spacc.at[pl.ds(s * BINS, 24)], gs_hbm.at[b, h])


def _tc_body(gs_ref, qp_ref, cp_ref, binmat_ref,
             w1a_ref, wfa_ref, hb1_ref, va_ref,
             b1a_ref, bfa_ref, hb1b_ref, vab_ref,
             w1b_ref, b1b_ref, wfb_ref, bfb_ref, vbw_ref, vbb_ref,
             out_ref):
    binmat = binmat_ref[...]
    aqs = jnp.dot(qp_ref[...].sum(axis=1), binmat,
                  preferred_element_type=jnp.float32)
    cnt = jnp.dot(cp_ref[...].sum(axis=1), binmat,
                  preferred_element_type=jnp.float32)
    gs = gs_ref[...].sum(axis=1)
    denom = jnp.maximum(cnt[:, :N_AG], 1.0)
    gsm = gs / denom[:, :, None]
    Bq = gs.shape[0]
    acc1 = jnp.zeros((Bq, 64), jnp.float32)
    accf = jnp.zeros((Bq, 64), jnp.float32)
    accb = jnp.zeros((Bq, EMBED_DIM), jnp.float32)
    accv = jnp.zeros((Bq, EMBED_DIM), jnp.float32)
    for a in range(N_AG):
        g = gsm[:, a, :]
        acc1 = acc1 + jnp.dot(g, w1a_ref[a], preferred_element_type=jnp.float32)
        accf = accf + jnp.dot(g, wfa_ref[a], preferred_element_type=jnp.float32)
        accb = accb + jnp.dot(g, hb1_ref[a], preferred_element_type=jnp.float32)
        accv = accv + jnp.dot(g, va_ref[a], preferred_element_type=jnp.float32)
    h1 = jnp.maximum(acc1 + b1a_ref[...], 0.0)
    hf = jnp.maximum(accf + bfa_ref[...], 0.0)
    b1v = accb + hb1b_ref[...]
    vah = jnp.maximum(accv + vab_ref[...], 0.0)
    w1 = jnp.abs(jnp.dot(h1, w1b_ref[...], preferred_element_type=jnp.float32)
                 + b1b_ref[...])
    wf = jnp.abs(jnp.dot(hf, wfb_ref[...], preferred_element_type=jnp.float32)
                 + bfb_ref[...])
    bf = jnp.sum(vah * vbw_ref[...], axis=1, keepdims=True) + vbb_ref[...]
    hp = b1v
    for k in range(N_AG):
        hp = hp + aqs[:, k:k + 1] * w1[:, k * EMBED_DIM:(k + 1) * EMBED_DIM]
    hid = jnp.where(hp > 0, hp, jnp.exp(jnp.minimum(hp, 0.0)) - 1.0)
    y = jnp.sum(hid * wf, axis=1, keepdims=True) + bf
    out_ref[...] = jnp.broadcast_to(y, out_ref.shape)


def kernel(qs_wt, aa_wt, aa_mut, agent_mask, mask, mutation_mask, emb_wt, emb_mut,
           w1a, b1a, w1b, b1b, wfa, bfa, wfb, bfb, hb1_w, hb1_b, va_w, va_b, vb_w, vb_b):
    B, L, n_ag = qs_wt.shape
    fea = emb_wt.shape[-1]
    am = agent_mask.astype(jnp.int32)
    m = mask.astype(jnp.int32)
    mm = mutation_mask.astype(jnp.int32)
    zeros = jnp.zeros((BINS, fea), jnp.float32)

    mesh = plsc.VectorSubcoreMesh(core_axis_name="c", subcore_axis_name="s")
    sc = pl.kernel(
        _sc_body,
        mesh=mesh,
        compiler_params=pltpu.CompilerParams(
            needs_layout_passes=False, use_tc_tiling_on_sc=False),
        out_type=(
            jax.ShapeDtypeStruct((B, L), jnp.float32),
            jax.ShapeDtypeStruct((B, 2, 24, fea), jnp.float32),
            jax.ShapeDtypeStruct((B, 2, BINS * 16), jnp.float32),
            jax.ShapeDtypeStruct((B, 2, BINS * 16), jnp.float32),
        ),
        scratch_types=[
            pltpu.VMEM((T * n_ag,), jnp.float32),
            pltpu.VMEM((T, fea), jnp.float32),
            pltpu.VMEM((T, fea), jnp.float32),
            pltpu.VMEM((T,), jnp.int32),
            pltpu.VMEM((T,), jnp.int32),
            pltpu.VMEM((T,), jnp.int32),
            pltpu.VMEM((T,), jnp.int32),
            pltpu.VMEM((T,), jnp.int32),
            pltpu.VMEM((T,), jnp.int32),
            pltpu.VMEM((T,), jnp.int32),
            pltpu.VMEM((L // 2,), jnp.float32),
            pltpu.VMEM((BINS * 16,), jnp.float32),
            pltpu.VMEM((BINS * 16,), jnp.float32),
            pltpu.VMEM((BINS, fea), jnp.float32),
            pltpu.VMEM_SHARED((16 * BINS, fea), jnp.float32),
        ],
    )
    q_sele, gs_sum, q_parts, c_parts = sc(
        qs_wt.reshape(B, L * n_ag), aa_wt, aa_mut, am, m, mm, emb_wt, emb_mut, zeros)
    gs_sum = gs_sum[:, :, :N_AG, :]
    binmat = jnp.repeat(jnp.eye(BINS, dtype=jnp.float32), 16, axis=0)

    w1a3 = w1a.reshape(N_AG, fea, 64)
    wfa3 = wfa.reshape(N_AG, fea, 64)
    hb13 = hb1_w.reshape(N_AG, fea, EMBED_DIM)
    va3 = va_w.reshape(N_AG, fea, EMBED_DIM)
    y = pl.pallas_call(
        _tc_body,
        out_shape=jax.ShapeDtypeStruct((B, 128), jnp.float32),
    )(gs_sum, q_parts, c_parts, binmat,
      w1a3, wfa3, hb13, va3,
      b1a.reshape(1, 64), bfa.reshape(1, 64),
      hb1_b.reshape(1, EMBED_DIM), va_b.reshape(1, EMBED_DIM),
      w1b, b1b.reshape(1, n_ag * EMBED_DIM), wfb, bfb.reshape(1, EMBED_DIM),
      vb_w.reshape(1, EMBED_DIM), vb_b.reshape(1, 1))

    q_tot = y[:, :1].reshape(B, 1, 1)
    return (q_tot, q_sele)


# speedup vs baseline: 3.4020x; 1.3225x over previous
import functools

import jax
import jax.numpy as jnp
from jax import lax
from jax.experimental import pallas as pl
from jax.experimental.pallas import tpu as pltpu
from jax.experimental.pallas import tpu_sc as plsc

N_AG = 20
BINS = 32
T = 32
NBUF = 2
EMBED_DIM = 32


def _sc_body(qs_hbm, aaw_hbm, aam_hbm, am_hbm, m_hbm, mm_hbm, ew_hbm, em_hbm,
             zeros_hbm,
             qsel_hbm, gs_hbm, qparts_hbm, cparts_hbm,
             qs_buf, ew_buf, em_buf,
             aw_all, amut_all, amk_all, mk_all, mmk_all,
             eidxw_buf, eidxm_buf, qout_buf, qacc, cacc, spacc,
             gsem_w, gsem_m, ssem_w, ssem_m, qsem):
    B, L = aaw_hbm.shape
    s = lax.axis_index("s")
    c = lax.axis_index("c")
    lb = s // 2
    h = s % 2
    b = c * (B // 2) + lb
    tok_per_tile = L // 2
    tok_base = h * tok_per_tile
    nchunks = tok_per_tile // T

    for sl in range(NBUF):
        t0 = tok_base + sl * T
        pltpu.async_copy(ew_hbm.at[b, pl.ds(t0, T)], ew_buf.at[sl], gsem_w.at[sl])
        pltpu.async_copy(em_hbm.at[b, pl.ds(t0, T)], em_buf.at[sl], gsem_m.at[sl])
        pltpu.async_copy(qs_hbm.at[b, pl.ds(t0 * N_AG, T * N_AG)],
                         qs_buf.at[sl], qsem.at[sl])

    pltpu.sync_copy(aaw_hbm.at[b, pl.ds(tok_base, tok_per_tile)], aw_all)
    pltpu.sync_copy(aam_hbm.at[b, pl.ds(tok_base, tok_per_tile)], amut_all)
    pltpu.sync_copy(am_hbm.at[b, pl.ds(tok_base, tok_per_tile)], amk_all)
    pltpu.sync_copy(m_hbm.at[b, pl.ds(tok_base, tok_per_tile)], mk_all)
    pltpu.sync_copy(mm_hbm.at[b, pl.ds(tok_base, tok_per_tile)], mmk_all)

    for j in range(BINS):
        qacc[pl.ds(j * 16, 16)] = jnp.zeros((16,), jnp.float32)
        cacc[pl.ds(j * 16, 16)] = jnp.zeros((16,), jnp.float32)

    pltpu.sync_copy(zeros_hbm, spacc.at[pl.ds(s * BINS, BINS)])

    lanes = lax.iota(jnp.int32, 16)
    sbase = s * BINS
    trash = sbase + N_AG + (lanes & 7)

    def grp(g, _):
        for sl in range(NBUF):
            i = g * NBUF + sl
            t0 = tok_base + i * T
            pltpu.make_async_copy(ew_hbm.at[b, pl.ds(t0, T)], ew_buf.at[sl],
                                  gsem_w.at[sl]).wait()
            pltpu.make_async_copy(em_hbm.at[b, pl.ds(t0, T)], em_buf.at[sl],
                                  gsem_m.at[sl]).wait()
            pltpu.make_async_copy(qs_hbm.at[b, pl.ds(t0 * N_AG, T * N_AG)],
                                  qs_buf.at[sl], qsem.at[sl]).wait()
            for k in range(T // 16):
                toks = pl.ds(i * T + k * 16, 16)
                aw = aw_all[toks]
                amut = amut_all[toks]
                valid = (amk_all[toks] & mk_all[toks]) != 0
                sel = jnp.where(mmk_all[toks] != 0, amut, aw)
                q = plsc.load_gather(
                    qs_buf.at[sl], [(k * 16 + lanes) * N_AG + sel])
                qv = jnp.where(valid, q, 0.0)
                qout_buf[toks] = qv
                plsc.addupdate_scatter(qacc, [aw * 16 + lanes], qv)
                ones = jnp.where(valid, 1.0, 0.0)
                plsc.addupdate_scatter(cacc, [aw * 16 + lanes], ones)
                plsc.addupdate_scatter(cacc, [amut * 16 + lanes], ones)
                eidxw_buf[sl, pl.ds(k * 16, 16)] = jnp.where(valid, sbase + aw, trash)
                eidxm_buf[sl, pl.ds(k * 16, 16)] = jnp.where(valid, sbase + amut, trash)
            pltpu.async_copy(ew_buf.at[sl], spacc.at[eidxw_buf.at[sl]],
                             ssem_w.at[sl], add=True)
            pltpu.async_copy(em_buf.at[sl], spacc.at[eidxm_buf.at[sl]],
                             ssem_m.at[sl], add=True)
            pltpu.make_async_copy(ew_buf.at[sl], spacc.at[eidxw_buf.at[sl]],
                                  ssem_w.at[sl]).wait()
            pltpu.make_async_copy(em_buf.at[sl], spacc.at[eidxm_buf.at[sl]],
                                  ssem_m.at[sl]).wait()
            nt0 = t0 + NBUF * T

            @pl.when(i + NBUF < nchunks)
            def _():
                pltpu.async_copy(ew_hbm.at[b, pl.ds(nt0, T)], ew_buf.at[sl],
                                 gsem_w.at[sl])
                pltpu.async_copy(em_hbm.at[b, pl.ds(nt0, T)], em_buf.at[sl],
                                 gsem_m.at[sl])
                pltpu.async_copy(qs_hbm.at[b, pl.ds(nt0 * N_AG, T * N_AG)],
                                 qs_buf.at[sl], qsem.at[sl])
        return 0

    lax.fori_loop(0, nchunks // NBUF, grp, 0)

    pltpu.sync_copy(qout_buf, qsel_hbm.at[b, pl.ds(tok_base, tok_per_tile)])
    pltpu.sync_copy(qacc, qparts_hbm.at[b, h])
    pltpu.sync_copy(cacc, cparts_hbm.at[b, h])
    pltpu.sync_copy(spacc.at[pl.ds(s * BINS, 24)], gs_hbm.at[b, h])


def _tc_body(gs_ref, qp_ref, cp_ref, binmat_ref,
             w1a_ref, wfa_ref, hb1_ref, va_ref,
             b1a_ref, bfa_ref, hb1b_ref, vab_ref,
             w1b_ref, b1b_ref, wfb_ref, bfb_ref, vbw_ref, vbb_ref,
             out_ref):
    binmat = binmat_ref[...]
    aqs = jnp.dot(qp_ref[...].sum(axis=1), binmat,
                  preferred_element_type=jnp.float32)
    cnt = jnp.dot(cp_ref[...].sum(axis=1), binmat,
                  preferred_element_type=jnp.float32)
    gs = gs_ref[...].sum(axis=1)
    denom = jnp.maximum(cnt[:, :N_AG], 1.0)
    gsm = gs / denom[:, :, None]
    Bq = gs.shape[0]
    acc1 = jnp.zeros((Bq, 64), jnp.float32)
    accf = jnp.zeros((Bq, 64), jnp.float32)
    accb = jnp.zeros((Bq, EMBED_DIM), jnp.float32)
    accv = jnp.zeros((Bq, EMBED_DIM), jnp.float32)
    for a in range(N_AG):
        g = gsm[:, a, :]
        acc1 = acc1 + jnp.dot(g, w1a_ref[a], preferred_element_type=jnp.float32)
        accf = accf + jnp.dot(g, wfa_ref[a], preferred_element_type=jnp.float32)
        accb = accb + jnp.dot(g, hb1_ref[a], preferred_element_type=jnp.float32)
        accv = accv + jnp.dot(g, va_ref[a], preferred_element_type=jnp.float32)
    h1 = jnp.maximum(acc1 + b1a_ref[...], 0.0)
    hf = jnp.maximum(accf + bfa_ref[...], 0.0)
    b1v = accb + hb1b_ref[...]
    vah = jnp.maximum(accv + vab_ref[...], 0.0)
    w1 = jnp.abs(jnp.dot(h1, w1b_ref[...], preferred_element_type=jnp.float32)
                 + b1b_ref[...])
    wf = jnp.abs(jnp.dot(hf, wfb_ref[...], preferred_element_type=jnp.float32)
                 + bfb_ref[...])
    bf = jnp.sum(vah * vbw_ref[...], axis=1, keepdims=True) + vbb_ref[...]
    hp = b1v
    for k in range(N_AG):
        hp = hp + aqs[:, k:k + 1] * w1[:, k * EMBED_DIM:(k + 1) * EMBED_DIM]
    hid = jnp.where(hp > 0, hp, jnp.exp(jnp.minimum(hp, 0.0)) - 1.0)
    y = jnp.sum(hid * wf, axis=1, keepdims=True) + bf
    out_ref[...] = jnp.broadcast_to(y, out_ref.shape)


def kernel(qs_wt, aa_wt, aa_mut, agent_mask, mask, mutation_mask, emb_wt, emb_mut,
           w1a, b1a, w1b, b1b, wfa, bfa, wfb, bfb, hb1_w, hb1_b, va_w, va_b, vb_w, vb_b):
    B, L, n_ag = qs_wt.shape
    fea = emb_wt.shape[-1]
    am = agent_mask.astype(jnp.int32)
    m = mask.astype(jnp.int32)
    mm = mutation_mask.astype(jnp.int32)
    zeros = jnp.zeros((BINS, fea), jnp.float32)

    mesh = plsc.VectorSubcoreMesh(core_axis_name="c", subcore_axis_name="s")
    sc = pl.kernel(
        _sc_body,
        mesh=mesh,
        compiler_params=pltpu.CompilerParams(
            needs_layout_passes=False, use_tc_tiling_on_sc=False),
        out_type=(
            jax.ShapeDtypeStruct((B, L), jnp.float32),
            jax.ShapeDtypeStruct((B, 2, 24, fea), jnp.float32),
            jax.ShapeDtypeStruct((B, 2, BINS * 16), jnp.float32),
            jax.ShapeDtypeStruct((B, 2, BINS * 16), jnp.float32),
        ),
        scratch_types=[
            pltpu.VMEM((NBUF, T * n_ag), jnp.float32),
            pltpu.VMEM((NBUF, T, fea), jnp.float32),
            pltpu.VMEM((NBUF, T, fea), jnp.float32),
            pltpu.VMEM((L // 2,), jnp.int32),
            pltpu.VMEM((L // 2,), jnp.int32),
            pltpu.VMEM((L // 2,), jnp.int32),
            pltpu.VMEM((L // 2,), jnp.int32),
            pltpu.VMEM((L // 2,), jnp.int32),
            pltpu.VMEM((NBUF, T), jnp.int32),
            pltpu.VMEM((NBUF, T), jnp.int32),
            pltpu.VMEM((L // 2,), jnp.float32),
            pltpu.VMEM((BINS * 16,), jnp.float32),
            pltpu.VMEM((BINS * 16,), jnp.float32),
            pltpu.VMEM_SHARED((16 * BINS, fea), jnp.float32),
            pltpu.SemaphoreType.DMA((NBUF,)),
            pltpu.SemaphoreType.DMA((NBUF,)),
            pltpu.SemaphoreType.DMA((NBUF,)),
            pltpu.SemaphoreType.DMA((NBUF,)),
            pltpu.SemaphoreType.DMA((NBUF,)),
        ],
    )
    q_sele, gs_sum, q_parts, c_parts = sc(
        qs_wt.reshape(B, L * n_ag), aa_wt, aa_mut, am, m, mm, emb_wt, emb_mut, zeros)
    gs_sum = gs_sum[:, :, :N_AG, :]
    binmat = jnp.repeat(jnp.eye(BINS, dtype=jnp.float32), 16, axis=0)

    w1a3 = w1a.reshape(N_AG, fea, 64)
    wfa3 = wfa.reshape(N_AG, fea, 64)
    hb13 = hb1_w.reshape(N_AG, fea, EMBED_DIM)
    va3 = va_w.reshape(N_AG, fea, EMBED_DIM)
    y = pl.pallas_call(
        _tc_body,
        out_shape=jax.ShapeDtypeStruct((B, 128), jnp.float32),
    )(gs_sum, q_parts, c_parts, binmat,
      w1a3, wfa3, hb13, va3,
      b1a.reshape(1, 64), bfa.reshape(1, 64),
      hb1_b.reshape(1, EMBED_DIM), va_b.reshape(1, EMBED_DIM),
      w1b, b1b.reshape(1, n_ag * EMBED_DIM), wfb, bfb.reshape(1, EMBED_DIM),
      vb_w.reshape(1, EMBED_DIM), vb_b.reshape(1, 1))

    q_tot = y[:, :1].reshape(B, 1, 1)
    return (q_tot, q_sele)


# speedup vs baseline: 5.4174x; 1.5924x over previous
import functools

import jax
import jax.numpy as jnp
from jax import lax
from jax.experimental import pallas as pl
from jax.experimental.pallas import tpu as pltpu
from jax.experimental.pallas import tpu_sc as plsc

N_AG = 20
BINS = 32
T = 32
NBUF = 2
EMBED_DIM = 32


def _sc_body(qs_hbm, aaw_hbm, aam_hbm, am_hbm, m_hbm, mm_hbm, ew_hbm, em_hbm,
             zeros_hbm,
             qsel_hbm, gs_hbm, qparts_hbm, cparts_hbm,
             qs_buf, ew_buf, em_buf,
             aw_all, amut_all, amk_all, mk_all, mmk_all,
             binw_buf, binm_buf,
             eidxw_buf, eidxm_buf, qout_buf, qacc, cacc, spacc,
             gsem_w, gsem_m, ssem_w, ssem_m, qsem):
    B, L = aaw_hbm.shape
    s = lax.axis_index("s")
    c = lax.axis_index("c")
    lb = s // 2
    h = s % 2
    b = c * (B // 2) + lb
    tok_per_tile = L // 2
    tok_base = h * tok_per_tile
    nchunks = tok_per_tile // T

    for sl in range(NBUF):
        t0 = tok_base + sl * T
        r0 = (b * L + t0) * 4
        pltpu.async_copy(ew_hbm.at[pl.ds(r0, 4 * T)], ew_buf.at[sl], gsem_w.at[sl])
        pltpu.async_copy(em_hbm.at[pl.ds(r0, 4 * T)], em_buf.at[sl], gsem_m.at[sl])
        pltpu.async_copy(qs_hbm.at[b, pl.ds(t0 * N_AG, T * N_AG)],
                         qs_buf.at[sl], qsem.at[sl])

    pltpu.sync_copy(aaw_hbm.at[b, pl.ds(tok_base, tok_per_tile)], aw_all)
    pltpu.sync_copy(aam_hbm.at[b, pl.ds(tok_base, tok_per_tile)], amut_all)
    pltpu.sync_copy(am_hbm.at[b, pl.ds(tok_base, tok_per_tile)], amk_all)
    pltpu.sync_copy(m_hbm.at[b, pl.ds(tok_base, tok_per_tile)], mk_all)
    pltpu.sync_copy(mm_hbm.at[b, pl.ds(tok_base, tok_per_tile)], mmk_all)

    for j in range(BINS):
        qacc[pl.ds(j * 16, 16)] = jnp.zeros((16,), jnp.float32)
        cacc[pl.ds(j * 16, 16)] = jnp.zeros((16,), jnp.float32)

    pltpu.sync_copy(zeros_hbm, spacc.at[pl.ds(s * BINS * 4, BINS * 4)])

    lanes = lax.iota(jnp.int32, 16)
    lanes8 = lanes & 7
    hi = lanes >> 3
    sbase = s * BINS * 4
    trash = N_AG + lanes8

    def grp(g, _):
        for sl in range(NBUF):
            i = g * NBUF + sl
            t0 = tok_base + i * T
            r0 = (b * L + t0) * 4
            pltpu.make_async_copy(ew_hbm.at[pl.ds(r0, 4 * T)], ew_buf.at[sl],
                                  gsem_w.at[sl]).wait()
            pltpu.make_async_copy(em_hbm.at[pl.ds(r0, 4 * T)], em_buf.at[sl],
                                  gsem_m.at[sl]).wait()
            pltpu.make_async_copy(qs_hbm.at[b, pl.ds(t0 * N_AG, T * N_AG)],
                                  qs_buf.at[sl], qsem.at[sl]).wait()
            for k in range(T // 16):
                toks = pl.ds(i * T + k * 16, 16)
                aw = aw_all[toks]
                amut = amut_all[toks]
                valid = (amk_all[toks] & mk_all[toks]) != 0
                sel = jnp.where(mmk_all[toks] != 0, amut, aw)
                q = plsc.load_gather(
                    qs_buf.at[sl], [(k * 16 + lanes) * N_AG + sel])
                qv = jnp.where(valid, q, 0.0)
                qout_buf[toks] = qv
                plsc.addupdate_scatter(qacc, [aw * 16 + lanes], qv)
                ones = jnp.where(valid, 1.0, 0.0)
                plsc.addupdate_scatter(cacc, [aw * 16 + lanes], ones)
                plsc.addupdate_scatter(cacc, [amut * 16 + lanes], ones)
                binw_buf[pl.ds(k * 16, 16)] = jnp.where(valid, aw, trash)
                binm_buf[pl.ds(k * 16, 16)] = jnp.where(valid, amut, trash)
            for v in range(4 * T // 16):
                tvec = (v // 2) * 8 + lanes8
                cblk = (2 * v + hi) & 3
                bw = plsc.load_gather(binw_buf, [tvec])
                bm = plsc.load_gather(binm_buf, [tvec])
                eidxw_buf[sl, pl.ds(v * 16, 16)] = sbase + bw * 4 + cblk
                eidxm_buf[sl, pl.ds(v * 16, 16)] = sbase + bm * 4 + cblk
            pltpu.async_copy(ew_buf.at[sl], spacc.at[eidxw_buf.at[sl]],
                             ssem_w.at[sl], add=True)
            pltpu.async_copy(em_buf.at[sl], spacc.at[eidxm_buf.at[sl]],
                             ssem_m.at[sl], add=True)
            pltpu.make_async_copy(ew_buf.at[sl], spacc.at[eidxw_buf.at[sl]],
                                  ssem_w.at[sl]).wait()
            pltpu.make_async_copy(em_buf.at[sl], spacc.at[eidxm_buf.at[sl]],
                                  ssem_m.at[sl]).wait()
            nt0 = t0 + NBUF * T
            nr0 = r0 + NBUF * T * 4

            @pl.when(i + NBUF < nchunks)
            def _():
                pltpu.async_copy(ew_hbm.at[pl.ds(nr0, 4 * T)], ew_buf.at[sl],
                                 gsem_w.at[sl])
                pltpu.async_copy(em_hbm.at[pl.ds(nr0, 4 * T)], em_buf.at[sl],
                                 gsem_m.at[sl])
                pltpu.async_copy(qs_hbm.at[b, pl.ds(nt0 * N_AG, T * N_AG)],
                                 qs_buf.at[sl], qsem.at[sl])
        return 0

    lax.fori_loop(0, nchunks // NBUF, grp, 0)

    pltpu.sync_copy(qout_buf, qsel_hbm.at[b, pl.ds(tok_base, tok_per_tile)])
    pltpu.sync_copy(qacc, qparts_hbm.at[b, h])
    pltpu.sync_copy(cacc, cparts_hbm.at[b, h])
    pltpu.sync_copy(spacc.at[pl.ds(s * BINS * 4, N_AG * 4)],
                    gs_hbm.at[pl.ds((b * 2 + h) * N_AG * 4, N_AG * 4)])


def _tc_body(gs_ref, qp_ref, cp_ref, binmat_ref,
             w1a_ref, wfa_ref, hb1_ref, va_ref,
             b1a_ref, bfa_ref, hb1b_ref, vab_ref,
             w1b_ref, b1b_ref, wfb_ref, bfb_ref, vbw_ref, vbb_ref,
             out_ref):
    binmat = binmat_ref[...]
    aqs = jnp.dot(qp_ref[...].sum(axis=1), binmat,
                  preferred_element_type=jnp.float32)
    cnt = jnp.dot(cp_ref[...].sum(axis=1), binmat,
                  preferred_element_type=jnp.float32)
    gs = gs_ref[...].sum(axis=1)
    denom = jnp.maximum(cnt[:, :N_AG], 1.0)
    gsm = gs / denom[:, :, None]
    Bq = gs.shape[0]
    acc1 = jnp.zeros((Bq, 64), jnp.float32)
    accf = jnp.zeros((Bq, 64), jnp.float32)
    accb = jnp.zeros((Bq, EMBED_DIM), jnp.float32)
    accv = jnp.zeros((Bq, EMBED_DIM), jnp.float32)
    for a in range(N_AG):
        g = gsm[:, a, :]
        acc1 = acc1 + jnp.dot(g, w1a_ref[a], preferred_element_type=jnp.float32)
        accf = accf + jnp.dot(g, wfa_ref[a], preferred_element_type=jnp.float32)
        accb = accb + jnp.dot(g, hb1_ref[a], preferred_element_type=jnp.float32)
        accv = accv + jnp.dot(g, va_ref[a], preferred_element_type=jnp.float32)
    h1 = jnp.maximum(acc1 + b1a_ref[...], 0.0)
    hf = jnp.maximum(accf + bfa_ref[...], 0.0)
    b1v = accb + hb1b_ref[...]
    vah = jnp.maximum(accv + vab_ref[...], 0.0)
    w1 = jnp.abs(jnp.dot(h1, w1b_ref[...], preferred_element_type=jnp.float32)
                 + b1b_ref[...])
    wf = jnp.abs(jnp.dot(hf, wfb_ref[...], preferred_element_type=jnp.float32)
                 + bfb_ref[...])
    bf = jnp.sum(vah * vbw_ref[...], axis=1, keepdims=True) + vbb_ref[...]
    hp = b1v
    for k in range(N_AG):
        hp = hp + aqs[:, k:k + 1] * w1[:, k * EMBED_DIM:(k + 1) * EMBED_DIM]
    hid = jnp.where(hp > 0, hp, jnp.exp(jnp.minimum(hp, 0.0)) - 1.0)
    y = jnp.sum(hid * wf, axis=1, keepdims=True) + bf
    out_ref[...] = jnp.broadcast_to(y, out_ref.shape)


def kernel(qs_wt, aa_wt, aa_mut, agent_mask, mask, mutation_mask, emb_wt, emb_mut,
           w1a, b1a, w1b, b1b, wfa, bfa, wfb, bfb, hb1_w, hb1_b, va_w, va_b, vb_w, vb_b):
    B, L, n_ag = qs_wt.shape
    fea = emb_wt.shape[-1]
    am = agent_mask.astype(jnp.int32)
    m = mask.astype(jnp.int32)
    mm = mutation_mask.astype(jnp.int32)
    zeros = jnp.zeros((BINS * 4, 128), jnp.float32)
    ewv = emb_wt.reshape(B, L // 8, 8, 4, 128).transpose(0, 1, 3, 2, 4)
    ewv = ewv.reshape(B * L * 4, 128)
    emv = emb_mut.reshape(B, L // 8, 8, 4, 128).transpose(0, 1, 3, 2, 4)
    emv = emv.reshape(B * L * 4, 128)

    mesh = plsc.VectorSubcoreMesh(core_axis_name="c", subcore_axis_name="s")
    sc = pl.kernel(
        _sc_body,
        mesh=mesh,
        compiler_params=pltpu.CompilerParams(
            needs_layout_passes=False, use_tc_tiling_on_sc=False),
        out_type=(
            jax.ShapeDtypeStruct((B, L), jnp.float32),
            jax.ShapeDtypeStruct((B * 2 * N_AG * 4, 128), jnp.float32),
            jax.ShapeDtypeStruct((B, 2, BINS * 16), jnp.float32),
            jax.ShapeDtypeStruct((B, 2, BINS * 16), jnp.float32),
        ),
        scratch_types=[
            pltpu.VMEM((NBUF, T * n_ag), jnp.float32),
            pltpu.VMEM((NBUF, 4 * T, 128), jnp.float32),
            pltpu.VMEM((NBUF, 4 * T, 128), jnp.float32),
            pltpu.VMEM((L // 2,), jnp.int32),
            pltpu.VMEM((L // 2,), jnp.int32),
            pltpu.VMEM((L // 2,), jnp.int32),
            pltpu.VMEM((L // 2,), jnp.int32),
            pltpu.VMEM((L // 2,), jnp.int32),
            pltpu.VMEM((128,), jnp.int32),
            pltpu.VMEM((128,), jnp.int32),
            pltpu.VMEM((NBUF, 4 * T), jnp.int32),
            pltpu.VMEM((NBUF, 4 * T), jnp.int32),
            pltpu.VMEM((L // 2,), jnp.float32),
            pltpu.VMEM((BINS * 16,), jnp.float32),
            pltpu.VMEM((BINS * 16,), jnp.float32),
            pltpu.VMEM_SHARED((16 * BINS * 4, 128), jnp.float32),
            pltpu.SemaphoreType.DMA((NBUF,)),
            pltpu.SemaphoreType.DMA((NBUF,)),
            pltpu.SemaphoreType.DMA((NBUF,)),
            pltpu.SemaphoreType.DMA((NBUF,)),
            pltpu.SemaphoreType.DMA((NBUF,)),
        ],
    )
    q_sele, gs_sum, q_parts, c_parts = sc(
        qs_wt.reshape(B, L * n_ag), aa_wt, aa_mut, am, m, mm, ewv, emv, zeros)
    gs_sum = gs_sum.reshape(B, 2, N_AG, fea)
    binmat = jnp.repeat(jnp.eye(BINS, dtype=jnp.float32), 16, axis=0)

    w1a3 = w1a.reshape(N_AG, fea, 64)
    wfa3 = wfa.reshape(N_AG, fea, 64)
    hb13 = hb1_w.reshape(N_AG, fea, EMBED_DIM)
    va3 = va_w.reshape(N_AG, fea, EMBED_DIM)
    y = pl.pallas_call(
        _tc_body,
        out_shape=jax.ShapeDtypeStruct((B, 128), jnp.float32),
    )(gs_sum, q_parts, c_parts, binmat,
      w1a3, wfa3, hb13, va3,
      b1a.reshape(1, 64), bfa.reshape(1, 64),
      hb1_b.reshape(1, EMBED_DIM), va_b.reshape(1, EMBED_DIM),
      w1b, b1b.reshape(1, n_ag * EMBED_DIM), wfb, bfb.reshape(1, EMBED_DIM),
      vb_w.reshape(1, EMBED_DIM), vb_b.reshape(1, 1))

    q_tot = y[:, :1].reshape(B, 1, 1)
    return (q_tot, q_sele)


# speedup vs baseline: 6.7177x; 1.2400x over previous
import functools

import jax
import jax.numpy as jnp
from jax import lax
from jax.experimental import pallas as pl
from jax.experimental.pallas import tpu as pltpu
from jax.experimental.pallas import tpu_sc as plsc

N_AG = 20
BINS = 32
T = 16
NBUF = 2
EMBED_DIM = 32


def _sc_body(qs_hbm, aaw_hbm, aam_hbm, am_hbm, m_hbm, mm_hbm, ew_hbm, em_hbm,
             zeros_hbm,
             qsel_hbm, gs_hbm, qparts_hbm, cparts_hbm,
             qs_buf, ew_buf, em_buf,
             aw_all, amut_all, amk_all, mk_all, mmk_all,
             binw_buf, binm_buf,
             eidxw_buf, eidxm_buf, qout_buf, qacc, cacc, spacc,
             gsem_w, gsem_m, ssem_w, ssem_m, qsem):
    B, L = aaw_hbm.shape
    s = lax.axis_index("s")
    c = lax.axis_index("c")
    lb = s // 2
    h = s % 2
    b = c * (B // 2) + lb
    tok_per_tile = L // 2
    tok_base = h * tok_per_tile
    nchunks = tok_per_tile // T

    pltpu.async_copy(qs_hbm.at[:, b // 8, pl.ds(16 * h, 16), b % 8, :],
                     qs_buf, qsem)
    for sl in range(NBUF):
        t0 = tok_base + sl * T
        r0 = (b * L + t0) * 4
        pltpu.async_copy(ew_hbm.at[pl.ds(r0, 4 * T)], ew_buf.at[sl], gsem_w.at[sl])
        pltpu.async_copy(em_hbm.at[pl.ds(r0, 4 * T)], em_buf.at[sl], gsem_m.at[sl])

    pltpu.sync_copy(aaw_hbm.at[b, pl.ds(tok_base, tok_per_tile)], aw_all)
    pltpu.sync_copy(aam_hbm.at[b, pl.ds(tok_base, tok_per_tile)], amut_all)
    pltpu.sync_copy(am_hbm.at[b, pl.ds(tok_base, tok_per_tile)], amk_all)
    pltpu.sync_copy(m_hbm.at[b, pl.ds(tok_base, tok_per_tile)], mk_all)
    pltpu.sync_copy(mm_hbm.at[b, pl.ds(tok_base, tok_per_tile)], mmk_all)

    for j in range(BINS):
        qacc[pl.ds(j * 16, 16)] = jnp.zeros((16,), jnp.float32)
        cacc[pl.ds(j * 16, 16)] = jnp.zeros((16,), jnp.float32)

    pltpu.sync_copy(zeros_hbm, spacc.at[pl.ds(s * BINS * 4, BINS * 4)])

    lanes = lax.iota(jnp.int32, 16)
    lanes8 = lanes & 7
    hi = lanes >> 3
    sbase = s * BINS * 4
    trash = N_AG + lanes8
    pltpu.make_async_copy(
        qs_hbm.at[:, b // 8, pl.ds(16 * h, 16), b % 8, :], qs_buf, qsem).wait()

    def grp(g, _):
        for sl in range(NBUF):
            i = g * NBUF + sl
            t0 = tok_base + i * T
            r0 = (b * L + t0) * 4
            pltpu.make_async_copy(ew_hbm.at[pl.ds(r0, 4 * T)], ew_buf.at[sl],
                                  gsem_w.at[sl]).wait()
            pltpu.make_async_copy(em_hbm.at[pl.ds(r0, 4 * T)], em_buf.at[sl],
                                  gsem_m.at[sl]).wait()
            for k in range(T // 16):
                toks = pl.ds(i * T + k * 16, 16)
                aw = aw_all[toks]
                amut = amut_all[toks]
                valid = (amk_all[toks] & mk_all[toks]) != 0
                sel = jnp.where(mmk_all[toks] != 0, amut, aw)
                tloc = i * T + k * 16 + lanes
                q = plsc.load_gather(
                    qs_buf, [sel, tloc >> 7, tloc & 127])
                qv = jnp.where(valid, q, 0.0)
                qout_buf[toks] = qv
                plsc.addupdate_scatter(qacc, [aw * 16 + lanes], qv)
                ones = jnp.where(valid, 1.0, 0.0)
                plsc.addupdate_scatter(cacc, [aw * 16 + lanes], ones)
                plsc.addupdate_scatter(cacc, [amut * 16 + lanes], ones)
                binw_buf[pl.ds(k * 16, 16)] = jnp.where(valid, aw, trash)
                binm_buf[pl.ds(k * 16, 16)] = jnp.where(valid, amut, trash)
            for v in range(4 * T // 16):
                tvec = (v // 2) * 8 + lanes8
                cblk = (2 * v + hi) & 3
                bw = plsc.load_gather(binw_buf, [tvec])
                bm = plsc.load_gather(binm_buf, [tvec])
                eidxw_buf[sl, pl.ds(v * 16, 16)] = sbase + bw * 4 + cblk
                eidxm_buf[sl, pl.ds(v * 16, 16)] = sbase + bm * 4 + cblk
            pltpu.async_copy(ew_buf.at[sl], spacc.at[eidxw_buf.at[sl]],
                             ssem_w.at[sl], add=True)
            pltpu.async_copy(em_buf.at[sl], spacc.at[eidxm_buf.at[sl]],
                             ssem_m.at[sl], add=True)
            pltpu.make_async_copy(ew_buf.at[sl], spacc.at[eidxw_buf.at[sl]],
                                  ssem_w.at[sl]).wait()
            pltpu.make_async_copy(em_buf.at[sl], spacc.at[eidxm_buf.at[sl]],
                                  ssem_m.at[sl]).wait()
            nr0 = r0 + NBUF * T * 4

            @pl.when(i + NBUF < nchunks)
            def _():
                pltpu.async_copy(ew_hbm.at[pl.ds(nr0, 4 * T)], ew_buf.at[sl],
                                 gsem_w.at[sl])
                pltpu.async_copy(em_hbm.at[pl.ds(nr0, 4 * T)], em_buf.at[sl],
                                 gsem_m.at[sl])
        return 0

    lax.fori_loop(0, nchunks // NBUF, grp, 0)

    pltpu.sync_copy(qout_buf, qsel_hbm.at[b, pl.ds(tok_base, tok_per_tile)])
    pltpu.sync_copy(qacc, qparts_hbm.at[b, h])
    pltpu.sync_copy(cacc, cparts_hbm.at[b, h])
    pltpu.sync_copy(spacc.at[pl.ds(s * BINS * 4, N_AG * 4)],
                    gs_hbm.at[pl.ds((b * 2 + h) * N_AG * 4, N_AG * 4)])


def _tc_body(gs_ref, qp_ref, cp_ref, binmat_ref,
             w1a_ref, wfa_ref, hb1_ref, va_ref,
             b1a_ref, bfa_ref, hb1b_ref, vab_ref,
             w1b_ref, b1b_ref, wfb_ref, bfb_ref, vbw_ref, vbb_ref,
             out_ref):
    binmat = binmat_ref[...]
    aqs = jnp.dot(qp_ref[...].sum(axis=1), binmat,
                  preferred_element_type=jnp.float32)
    cnt = jnp.dot(cp_ref[...].sum(axis=1), binmat,
                  preferred_element_type=jnp.float32)
    gs = gs_ref[...].sum(axis=1)
    denom = jnp.maximum(cnt[:, :N_AG], 1.0)
    gsm = gs / denom[:, :, None]
    Bq = gs.shape[0]
    acc1 = jnp.zeros((Bq, 64), jnp.float32)
    accf = jnp.zeros((Bq, 64), jnp.float32)
    accb = jnp.zeros((Bq, EMBED_DIM), jnp.float32)
    accv = jnp.zeros((Bq, EMBED_DIM), jnp.float32)
    for a in range(N_AG):
        g = gsm[:, a, :]
        acc1 = acc1 + jnp.dot(g, w1a_ref[a], preferred_element_type=jnp.float32)
        accf = accf + jnp.dot(g, wfa_ref[a], preferred_element_type=jnp.float32)
        accb = accb + jnp.dot(g, hb1_ref[a], preferred_element_type=jnp.float32)
        accv = accv + jnp.dot(g, va_ref[a], preferred_element_type=jnp.float32)
    h1 = jnp.maximum(acc1 + b1a_ref[...], 0.0)
    hf = jnp.maximum(accf + bfa_ref[...], 0.0)
    b1v = accb + hb1b_ref[...]
    vah = jnp.maximum(accv + vab_ref[...], 0.0)
    w1 = jnp.abs(jnp.dot(h1, w1b_ref[...], preferred_element_type=jnp.float32)
                 + b1b_ref[...])
    wf = jnp.abs(jnp.dot(hf, wfb_ref[...], preferred_element_type=jnp.float32)
                 + bfb_ref[...])
    bf = jnp.sum(vah * vbw_ref[...], axis=1, keepdims=True) + vbb_ref[...]
    hp = b1v
    for k in range(N_AG):
        hp = hp + aqs[:, k:k + 1] * w1[:, k * EMBED_DIM:(k + 1) * EMBED_DIM]
    hid = jnp.where(hp > 0, hp, jnp.exp(jnp.minimum(hp, 0.0)) - 1.0)
    y = jnp.sum(hid * wf, axis=1, keepdims=True) + bf
    out_ref[...] = jnp.broadcast_to(y, out_ref.shape)


def kernel(qs_wt, aa_wt, aa_mut, agent_mask, mask, mutation_mask, emb_wt, emb_mut,
           w1a, b1a, w1b, b1b, wfa, bfa, wfb, bfb, hb1_w, hb1_b, va_w, va_b, vb_w, vb_b):
    B, L, n_ag = qs_wt.shape
    fea = emb_wt.shape[-1]
    am = agent_mask.astype(jnp.int32)
    m = mask.astype(jnp.int32)
    mm = mutation_mask.astype(jnp.int32)
    zeros = jnp.zeros((BINS * 4, 128), jnp.float32)
    ewv = emb_wt.reshape(B, L // 8, 8, 4, 128).transpose(0, 1, 3, 2, 4)
    ewv = ewv.reshape(B * L * 4, 128)
    emv = emb_mut.reshape(B, L // 8, 8, 4, 128).transpose(0, 1, 3, 2, 4)
    emv = emv.reshape(B * L * 4, 128)
    qsv = qs_wt.transpose(2, 0, 1).reshape(n_ag, B // 8, 8, L // 128, 128)
    qsv = qsv.transpose(0, 1, 3, 2, 4)

    mesh = plsc.VectorSubcoreMesh(core_axis_name="c", subcore_axis_name="s")
    sc = pl.kernel(
        _sc_body,
        mesh=mesh,
        compiler_params=pltpu.CompilerParams(
            needs_layout_passes=False, use_tc_tiling_on_sc=False),
        out_type=(
            jax.ShapeDtypeStruct((B, L), jnp.float32),
            jax.ShapeDtypeStruct((B * 2 * N_AG * 4, 128), jnp.float32),
            jax.ShapeDtypeStruct((B, 2, BINS * 16), jnp.float32),
            jax.ShapeDtypeStruct((B, 2, BINS * 16), jnp.float32),
        ),
        scratch_types=[
            pltpu.VMEM((N_AG, 16, 128), jnp.float32),
            pltpu.VMEM((NBUF, 4 * T, 128), jnp.float32),
            pltpu.VMEM((NBUF, 4 * T, 128), jnp.float32),
            pltpu.VMEM((L // 2,), jnp.int32),
            pltpu.VMEM((L // 2,), jnp.int32),
            pltpu.VMEM((L // 2,), jnp.int32),
            pltpu.VMEM((L // 2,), jnp.int32),
            pltpu.VMEM((L // 2,), jnp.int32),
            pltpu.VMEM((128,), jnp.int32),
            pltpu.VMEM((128,), jnp.int32),
            pltpu.VMEM((NBUF, 4 * T), jnp.int32),
            pltpu.VMEM((NBUF, 4 * T), jnp.int32),
            pltpu.VMEM((L // 2,), jnp.float32),
            pltpu.VMEM((BINS * 16,), jnp.float32),
            pltpu.VMEM((BINS * 16,), jnp.float32),
            pltpu.VMEM_SHARED((16 * BINS * 4, 128), jnp.float32),
            pltpu.SemaphoreType.DMA((NBUF,)),
            pltpu.SemaphoreType.DMA((NBUF,)),
            pltpu.SemaphoreType.DMA((NBUF,)),
            pltpu.SemaphoreType.DMA((NBUF,)),
            pltpu.SemaphoreType.DMA,
        ],
    )
    q_sele, gs_sum, q_parts, c_parts = sc(
        qsv, aa_wt, aa_mut, am, m, mm, ewv, emv, zeros)
    gs_sum = gs_sum.reshape(B, 2, N_AG, fea)
    binmat = jnp.repeat(jnp.eye(BINS, dtype=jnp.float32), 16, axis=0)

    w1a3 = w1a.reshape(N_AG, fea, 64)
    wfa3 = wfa.reshape(N_AG, fea, 64)
    hb13 = hb1_w.reshape(N_AG, fea, EMBED_DIM)
    va3 = va_w.reshape(N_AG, fea, EMBED_DIM)
    y = pl.pallas_call(
        _tc_body,
        out_shape=jax.ShapeDtypeStruct((B, 128), jnp.float32),
    )(gs_sum, q_parts, c_parts, binmat,
      w1a3, wfa3, hb13, va3,
      b1a.reshape(1, 64), bfa.reshape(1, 64),
      hb1_b.reshape(1, EMBED_DIM), va_b.reshape(1, EMBED_DIM),
      w1b, b1b.reshape(1, n_ag * EMBED_DIM), wfb, bfb.reshape(1, EMBED_DIM),
      vb_w.reshape(1, EMBED_DIM), vb_b.reshape(1, 1))

    q_tot = y[:, :1].reshape(B, 1, 1)
    return (q_tot, q_sele)


# speedup vs baseline: 6.7335x; 1.0024x over previous
import functools

import jax
import jax.numpy as jnp
from jax import lax
from jax.experimental import pallas as pl
from jax.experimental.pallas import tpu as pltpu
from jax.experimental.pallas import tpu_sc as plsc

N_AG = 20
BINS = 32
T = 16
NBUF = 2
EMBED_DIM = 32


def _sc_body(qs_hbm, aaw_hbm, aam_hbm, am_hbm, m_hbm, mm_hbm, ew_hbm, em_hbm,
             zeros_hbm,
             qsel_hbm, gs_hbm, qparts_hbm, cparts_hbm,
             qs_buf, ew_buf, em_buf,
             aw_all, amut_all, amk_all, mk_all, mmk_all,
             binw_buf, binm_buf,
             eidxw_buf, eidxm_buf, qout_buf, qacc, cacc, spacc,
             gsem_w, gsem_m, ssem_w, ssem_m, qsem):
    B, L = aaw_hbm.shape
    s = lax.axis_index("s")
    c = lax.axis_index("c")
    lb = s // 2
    h = s % 2
    b = c * (B // 2) + lb
    tok_per_tile = L // 2
    tok_base = h * tok_per_tile
    nchunks = tok_per_tile // T

    pltpu.async_copy(qs_hbm.at[:, b // 8, pl.ds(16 * h, 16), b % 8, :],
                     qs_buf, qsem)
    for sl in range(NBUF):
        t0 = tok_base + sl * T
        r0 = (b * L + t0) * 4
        pltpu.async_copy(ew_hbm.at[pl.ds(r0, 4 * T)], ew_buf.at[sl], gsem_w.at[sl])
        pltpu.async_copy(em_hbm.at[pl.ds(r0, 4 * T)], em_buf.at[sl], gsem_m.at[sl])

    pltpu.sync_copy(aaw_hbm.at[b, pl.ds(tok_base, tok_per_tile)], aw_all)
    pltpu.sync_copy(aam_hbm.at[b, pl.ds(tok_base, tok_per_tile)], amut_all)
    pltpu.sync_copy(am_hbm.at[b, pl.ds(tok_base, tok_per_tile)], amk_all)
    pltpu.sync_copy(m_hbm.at[b, pl.ds(tok_base, tok_per_tile)], mk_all)
    pltpu.sync_copy(mm_hbm.at[b, pl.ds(tok_base, tok_per_tile)], mmk_all)

    for j in range(BINS):
        qacc[pl.ds(j * 16, 16)] = jnp.zeros((16,), jnp.float32)
        cacc[pl.ds(j * 16, 16)] = jnp.zeros((16,), jnp.float32)

    pltpu.sync_copy(zeros_hbm, spacc.at[pl.ds(s * BINS * 4, BINS * 4)])

    lanes = lax.iota(jnp.int32, 16)
    lanes8 = lanes & 7
    hi = lanes >> 3
    sbase = s * BINS * 4
    trash = N_AG + lanes8
    pltpu.make_async_copy(
        qs_hbm.at[:, b // 8, pl.ds(16 * h, 16), b % 8, :], qs_buf, qsem).wait()

    def grp(g, _):
        for sl in range(NBUF):
            i = g * NBUF + sl
            t0 = tok_base + i * T
            r0 = (b * L + t0) * 4
            for k in range(T // 16):
                toks = pl.ds(i * T + k * 16, 16)
                aw = aw_all[toks]
                amut = amut_all[toks]
                valid = (amk_all[toks] & mk_all[toks]) != 0
                sel = jnp.where(mmk_all[toks] != 0, amut, aw)
                tloc = i * T + k * 16 + lanes
                q = plsc.load_gather(
                    qs_buf, [sel, tloc >> 7, tloc & 127])
                qv = jnp.where(valid, q, 0.0)
                qout_buf[toks] = qv
                plsc.addupdate_scatter(qacc, [aw * 16 + lanes], qv)
                ones = jnp.where(valid, 1.0, 0.0)
                plsc.addupdate_scatter(cacc, [aw * 16 + lanes], ones)
                plsc.addupdate_scatter(cacc, [amut * 16 + lanes], ones)
                binw_buf[pl.ds(k * 16, 16)] = jnp.where(valid, aw, trash)
                binm_buf[pl.ds(k * 16, 16)] = jnp.where(valid, amut, trash)
            for v in range(4 * T // 16):
                tvec = (v // 2) * 8 + lanes8
                cblk = (2 * v + hi) & 3
                bw = plsc.load_gather(binw_buf, [tvec])
                bm = plsc.load_gather(binm_buf, [tvec])
                eidxw_buf[sl, pl.ds(v * 16, 16)] = sbase + bw * 4 + cblk
                eidxm_buf[sl, pl.ds(v * 16, 16)] = sbase + bm * 4 + cblk
            pltpu.make_async_copy(ew_hbm.at[pl.ds(r0, 4 * T)], ew_buf.at[sl],
                                  gsem_w.at[sl]).wait()
            pltpu.make_async_copy(em_hbm.at[pl.ds(r0, 4 * T)], em_buf.at[sl],
                                  gsem_m.at[sl]).wait()
            pltpu.async_copy(ew_buf.at[sl], spacc.at[eidxw_buf.at[sl]],
                             ssem_w.at[sl], add=True)
            pltpu.async_copy(em_buf.at[sl], spacc.at[eidxm_buf.at[sl]],
                             ssem_m.at[sl], add=True)
            pltpu.make_async_copy(ew_buf.at[sl], spacc.at[eidxw_buf.at[sl]],
                                  ssem_w.at[sl]).wait()
            pltpu.make_async_copy(em_buf.at[sl], spacc.at[eidxm_buf.at[sl]],
                                  ssem_m.at[sl]).wait()
            nr0 = r0 + NBUF * T * 4

            @pl.when(i + NBUF < nchunks)
            def _():
                pltpu.async_copy(ew_hbm.at[pl.ds(nr0, 4 * T)], ew_buf.at[sl],
                                 gsem_w.at[sl])
                pltpu.async_copy(em_hbm.at[pl.ds(nr0, 4 * T)], em_buf.at[sl],
                                 gsem_m.at[sl])
        return 0

    lax.fori_loop(0, nchunks // NBUF, grp, 0)

    pltpu.sync_copy(qout_buf, qsel_hbm.at[b, pl.ds(tok_base, tok_per_tile)])
    pltpu.sync_copy(qacc, qparts_hbm.at[b, h])
    pltpu.sync_copy(cacc, cparts_hbm.at[b, h])
    pltpu.sync_copy(spacc.at[pl.ds(s * BINS * 4, N_AG * 4)],
                    gs_hbm.at[pl.ds((b * 2 + h) * N_AG * 4, N_AG * 4)])


def _tc_body(gs_ref, qp_ref, cp_ref, binmat_ref,
             w1a_ref, wfa_ref, hb1_ref, va_ref,
             b1a_ref, bfa_ref, hb1b_ref, vab_ref,
             w1b_ref, b1b_ref, wfb_ref, bfb_ref, vbw_ref, vbb_ref,
             out_ref):
    binmat = binmat_ref[...]
    aqs = jnp.dot(qp_ref[...].sum(axis=1), binmat,
                  preferred_element_type=jnp.float32)
    cnt = jnp.dot(cp_ref[...].sum(axis=1), binmat,
                  preferred_element_type=jnp.float32)
    gs = gs_ref[...].sum(axis=1)
    denom = jnp.maximum(cnt[:, :N_AG], 1.0)
    gsm = gs / denom[:, :, None]
    Bq = gs.shape[0]
    acc1 = jnp.zeros((Bq, 64), jnp.float32)
    accf = jnp.zeros((Bq, 64), jnp.float32)
    accb = jnp.zeros((Bq, EMBED_DIM), jnp.float32)
    accv = jnp.zeros((Bq, EMBED_DIM), jnp.float32)
    for a in range(N_AG):
        g = gsm[:, a, :]
        acc1 = acc1 + jnp.dot(g, w1a_ref[a], preferred_element_type=jnp.float32)
        accf = accf + jnp.dot(g, wfa_ref[a], preferred_element_type=jnp.float32)
        accb = accb + jnp.dot(g, hb1_ref[a], preferred_element_type=jnp.float32)
        accv = accv + jnp.dot(g, va_ref[a], preferred_element_type=jnp.float32)
    h1 = jnp.maximum(acc1 + b1a_ref[...], 0.0)
    hf = jnp.maximum(accf + bfa_ref[...], 0.0)
    b1v = accb + hb1b_ref[...]
    vah = jnp.maximum(accv + vab_ref[...], 0.0)
    w1 = jnp.abs(jnp.dot(h1, w1b_ref[...], preferred_element_type=jnp.float32)
                 + b1b_ref[...])
    wf = jnp.abs(jnp.dot(hf, wfb_ref[...], preferred_element_type=jnp.float32)
                 + bfb_ref[...])
    bf = jnp.sum(vah * vbw_ref[...], axis=1, keepdims=True) + vbb_ref[...]
    hp = b1v
    for k in range(N_AG):
        hp = hp + aqs[:, k:k + 1] * w1[:, k * EMBED_DIM:(k + 1) * EMBED_DIM]
    hid = jnp.where(hp > 0, hp, jnp.exp(jnp.minimum(hp, 0.0)) - 1.0)
    y = jnp.sum(hid * wf, axis=1, keepdims=True) + bf
    out_ref[...] = jnp.broadcast_to(y, out_ref.shape)


def kernel(qs_wt, aa_wt, aa_mut, agent_mask, mask, mutation_mask, emb_wt, emb_mut,
           w1a, b1a, w1b, b1b, wfa, bfa, wfb, bfb, hb1_w, hb1_b, va_w, va_b, vb_w, vb_b):
    B, L, n_ag = qs_wt.shape
    fea = emb_wt.shape[-1]
    am = agent_mask.astype(jnp.int32)
    m = mask.astype(jnp.int32)
    mm = mutation_mask.astype(jnp.int32)
    zeros = jnp.zeros((BINS * 4, 128), jnp.float32)
    ewv = emb_wt.reshape(B, L // 8, 8, 4, 128).transpose(0, 1, 3, 2, 4)
    ewv = ewv.reshape(B * L * 4, 128)
    emv = emb_mut.reshape(B, L // 8, 8, 4, 128).transpose(0, 1, 3, 2, 4)
    emv = emv.reshape(B * L * 4, 128)
    qsv = qs_wt.transpose(2, 0, 1).reshape(n_ag, B // 8, 8, L // 128, 128)
    qsv = qsv.transpose(0, 1, 3, 2, 4)

    mesh = plsc.VectorSubcoreMesh(core_axis_name="c", subcore_axis_name="s")
    sc = pl.kernel(
        _sc_body,
        mesh=mesh,
        compiler_params=pltpu.CompilerParams(
            needs_layout_passes=False, use_tc_tiling_on_sc=False),
        out_type=(
            jax.ShapeDtypeStruct((B, L), jnp.float32),
            jax.ShapeDtypeStruct((B * 2 * N_AG * 4, 128), jnp.float32),
            jax.ShapeDtypeStruct((B, 2, BINS * 16), jnp.float32),
            jax.ShapeDtypeStruct((B, 2, BINS * 16), jnp.float32),
        ),
        scratch_types=[
            pltpu.VMEM((N_AG, 16, 128), jnp.float32),
            pltpu.VMEM((NBUF, 4 * T, 128), jnp.float32),
            pltpu.VMEM((NBUF, 4 * T, 128), jnp.float32),
            pltpu.VMEM((L // 2,), jnp.int32),
            pltpu.VMEM((L // 2,), jnp.int32),
            pltpu.VMEM((L // 2,), jnp.int32),
            pltpu.VMEM((L // 2,), jnp.int32),
            pltpu.VMEM((L // 2,), jnp.int32),
            pltpu.VMEM((128,), jnp.int32),
            pltpu.VMEM((128,), jnp.int32),
            pltpu.VMEM((NBUF, 4 * T), jnp.int32),
            pltpu.VMEM((NBUF, 4 * T), jnp.int32),
            pltpu.VMEM((L // 2,), jnp.float32),
            pltpu.VMEM((BINS * 16,), jnp.float32),
            pltpu.VMEM((BINS * 16,), jnp.float32),
            pltpu.VMEM_SHARED((16 * BINS * 4, 128), jnp.float32),
            pltpu.SemaphoreType.DMA((NBUF,)),
            pltpu.SemaphoreType.DMA((NBUF,)),
            pltpu.SemaphoreType.DMA((NBUF,)),
            pltpu.SemaphoreType.DMA((NBUF,)),
            pltpu.SemaphoreType.DMA,
        ],
    )
    q_sele, gs_sum, q_parts, c_parts = sc(
        qsv, aa_wt, aa_mut, am, m, mm, ewv, emv, zeros)
    gs_sum = gs_sum.reshape(B, 2, N_AG, fea)
    binmat = jnp.repeat(jnp.eye(BINS, dtype=jnp.float32), 16, axis=0)

    w1a3 = w1a.reshape(N_AG, fea, 64)
    wfa3 = wfa.reshape(N_AG, fea, 64)
    hb13 = hb1_w.reshape(N_AG, fea, EMBED_DIM)
    va3 = va_w.reshape(N_AG, fea, EMBED_DIM)
    y = pl.pallas_call(
        _tc_body,
        out_shape=jax.ShapeDtypeStruct((B, 128), jnp.float32),
    )(gs_sum, q_parts, c_parts, binmat,
      w1a3, wfa3, hb13, va3,
      b1a.reshape(1, 64), bfa.reshape(1, 64),
      hb1_b.reshape(1, EMBED_DIM), va_b.reshape(1, EMBED_DIM),
      w1b, b1b.reshape(1, n_ag * EMBED_DIM), wfb, bfb.reshape(1, EMBED_DIM),
      vb_w.reshape(1, EMBED_DIM), vb_b.reshape(1, 1))

    q_tot = y[:, :1].reshape(B, 1, 1)
    return (q_tot, q_sele)
